# baseline (speedup 1.0000x reference)
import functools

import jax
import jax.numpy as jnp
from jax import lax
from jax.experimental import pallas as pl

N_DOCS = 100000
N_DOCS_PAD = 100352
DB = 2048
TOP_K = 100


def _scores_body(pq0_ref, pq1_ref, pd0_ref, pd1_ref, out_ref):
    i = pl.program_id(0)
    s0 = lax.dot_general(pq0_ref[...], pd0_ref[...],
                         (((1,), (1,)), ((), ())),
                         preferred_element_type=jnp.float32)
    s1 = lax.dot_general(pq1_ref[...], pd1_ref[...],
                         (((1,), (1,)), ((), ())),
                         preferred_element_type=jnp.float32)
    s = s0 + s1
    col = i * DB + lax.broadcasted_iota(jnp.int32, s.shape, 1)
    out_ref[...] = jnp.where(col < N_DOCS, s, jnp.float32(-1.0))


def _compute_scores(pq0, pq1, pd0, pd1):
    nq = pq0.shape[0]
    grid = (N_DOCS_PAD // DB,)
    return pl.pallas_call(
        _scores_body,
        grid=grid,
        in_specs=[
            pl.BlockSpec((nq, 32), lambda i: (0, 0)),
            pl.BlockSpec((nq, 32), lambda i: (0, 0)),
            pl.BlockSpec((DB, 32), lambda i: (i, 0)),
            pl.BlockSpec((DB, 32), lambda i: (i, 0)),
        ],
        out_specs=pl.BlockSpec((nq, DB), lambda i: (0, i)),
        out_shape=jax.ShapeDtypeStruct((nq, N_DOCS_PAD), jnp.float32),
    )(pq0, pq1, pd0, pd1)


def kernel(queries, documents, Wq, bq, Wd, bd):
    d_logits = documents @ Wd + bd
    q_logits = queries @ Wq + bq
    pq0 = jnp.exp(jax.nn.log_sigmoid(-q_logits))
    pq1 = jnp.exp(jax.nn.log_sigmoid(q_logits))
    pd0 = jnp.exp(jax.nn.log_sigmoid(-d_logits))
    pd1 = jnp.exp(jax.nn.log_sigmoid(d_logits))
    pad = N_DOCS_PAD - N_DOCS
    pd0 = jnp.pad(pd0, ((0, pad), (0, 0)))
    pd1 = jnp.pad(pd1, ((0, pad), (0, 0)))
    scores = _compute_scores(pq0, pq1, pd0, pd1)
    topk_scores, topk_indexes = jax.lax.top_k(scores, TOP_K)
    return topk_scores, topk_indexes


# speedup vs baseline: 4.6777x; 4.6777x over previous
import functools

import jax
import jax.numpy as jnp
from jax import lax
from jax.experimental import pallas as pl
from jax.experimental.pallas import tpu as pltpu

DB = 2048
TOP_K = 100
NSEL = 104
G = 16
CAND = NSEL * G

_INTERPRET = False
NEG = float("-inf")


def _scores_body(n_docs, pq0_ref, pq1_ref, pd0_ref, pd1_ref, sel_ref,
                 l0_ref, l1_ref, l2_ref):
    i = pl.program_id(0)
    s0 = lax.dot_general(pq0_ref[...], pd0_ref[...],
                         (((1,), (1,)), ((), ())),
                         preferred_element_type=jnp.float32)
    s1 = lax.dot_general(pq1_ref[...], pd1_ref[...],
                         (((1,), (1,)), ((), ())),
                         preferred_element_type=jnp.float32)
    s = s0 + s1
    col = i * DB + lax.broadcasted_iota(jnp.int32, s.shape, 1)
    s = jnp.where(col < n_docs, s, jnp.float32(-1.0))
    l0_ref[...] = s
    w = s
    for k in (1, 2, 4, 8):
        w = jnp.maximum(w, jnp.concatenate([w[:, k:], w[:, :k]], axis=1))
    w16 = w
    for k in (16, 32, 64, 128):
        w = jnp.maximum(w, jnp.concatenate([w[:, k:], w[:, :k]], axis=1))
    w256 = w
    nq = s.shape[0]
    l1 = lax.dot_general(w16, sel_ref[:, :DB // G],
                         (((1,), (0,)), ((), ())),
                         precision=lax.Precision.HIGHEST,
                         preferred_element_type=jnp.float32)
    l2 = lax.dot_general(w256, sel_ref[:, DB // G:],
                         (((1,), (0,)), ((), ())),
                         precision=lax.Precision.HIGHEST,
                         preferred_element_type=jnp.float32)
    l1_ref[...] = l1.reshape(1, nq, DB // G)
    l2_ref[...] = l2.reshape(1, nq, DB // 256)


def _compute_scores(pq0, pq1, pd0, pd1, sel, n_docs):
    nq = pq0.shape[0]
    ndp = pd0.shape[0]
    nb = ndp // DB
    grid = (nb,)
    return pl.pallas_call(
        functools.partial(_scores_body, n_docs),
        grid=grid,
        in_specs=[
            pl.BlockSpec((nq, 32), lambda i: (0, 0)),
            pl.BlockSpec((nq, 32), lambda i: (0, 0)),
            pl.BlockSpec((DB, 32), lambda i: (i, 0)),
            pl.BlockSpec((DB, 32), lambda i: (i, 0)),
            pl.BlockSpec((DB, DB // G + DB // 256), lambda i: (0, 0)),
        ],
        out_specs=[
            pl.BlockSpec((nq, DB), lambda i: (0, i)),
            pl.BlockSpec((1, nq, DB // G), lambda i: (i, 0, 0)),
            pl.BlockSpec((1, nq, DB // 256), lambda i: (i, 0, 0)),
        ],
        out_shape=[
            jax.ShapeDtypeStruct((nq, ndp), jnp.float32),
            jax.ShapeDtypeStruct((nb, nq, DB // G), jnp.float32),
            jax.ShapeDtypeStruct((nb, nq, DB // 256), jnp.float32),
        ],
        compiler_params=pltpu.CompilerParams(
            dimension_semantics=("arbitrary",)),
        interpret=_INTERPRET,
    )(pq0, pq1, pd0, pd1, sel)


def _extract_step(work_ref, ids, k, out_ids, out_vals, want_vals):
    row = work_ref[...]
    m = jnp.max(row, axis=1, keepdims=True)
    col = lax.broadcasted_iota(jnp.int32, row.shape, 1)
    cand_ids = ids if ids is not None else col
    sel = jnp.min(jnp.where(row == m, cand_ids, jnp.int32(2**30)),
                  axis=1, keepdims=True)
    hit = (row == m) & (cand_ids == sel)
    work_ref[...] = jnp.where(hit, NEG, row)
    kcol = lax.broadcasted_iota(jnp.int32, out_ids.shape, 1)
    out_ids = jnp.where(kcol == k, sel, out_ids)
    if want_vals:
        out_vals = jnp.where(kcol == k, m, out_vals)
    return out_ids, out_vals


def _extract_l2_body(vals_ref, ids_out_ref, work_ref):
    nq = vals_ref.shape[0]
    work_ref[...] = vals_ref[...]

    def step(k, carry):
        return _extract_step(work_ref, None, k, carry[0], carry[1], False)

    ids0 = jnp.zeros((nq, NSEL), jnp.int32)
    out_ids, _ = lax.fori_loop(0, NSEL, step, (ids0, ids0))
    ids_out_ref[...] = out_ids


def _extract_cand_body(n_out, want_vals, vals_ref, ids_ref, ids_out_ref,
                       *rest):
    if want_vals:
        vals_out_ref, work_ref = rest
    else:
        (work_ref,) = rest
    nq = vals_ref.shape[0]
    work_ref[...] = vals_ref[...]
    ids = ids_ref[...]

    def step(k, carry):
        return _extract_step(work_ref, ids, k, carry[0], carry[1], want_vals)

    ids0 = jnp.zeros((nq, n_out), jnp.int32)
    vals0 = jnp.zeros((nq, n_out), jnp.float32)
    out_ids, out_vals = lax.fori_loop(0, n_out, step, (ids0, vals0))
    ids_out_ref[...] = out_ids
    if want_vals:
        vals_out_ref[...] = out_vals


def _extract_l2(l2, qb):
    nq, w2 = l2.shape
    grid = (nq // qb,)
    return pl.pallas_call(
        _extract_l2_body,
        grid=grid,
        in_specs=[pl.BlockSpec((qb, w2), lambda i: (i, 0))],
        out_specs=pl.BlockSpec((qb, NSEL), lambda i: (i, 0)),
        out_shape=jax.ShapeDtypeStruct((nq, NSEL), jnp.int32),
        scratch_shapes=[pltpu.VMEM((qb, w2), jnp.float32)],
        compiler_params=pltpu.CompilerParams(
            dimension_semantics=("arbitrary",)),
        interpret=_INTERPRET,
    )(l2)


def _extract_cand(vals, ids, n_out, want_vals, qb):
    nq, w = vals.shape
    grid = (nq // qb,)
    out_specs = [pl.BlockSpec((qb, n_out), lambda i: (i, 0))]
    out_shape = [jax.ShapeDtypeStruct((nq, n_out), jnp.int32)]
    if want_vals:
        out_specs.append(pl.BlockSpec((qb, n_out), lambda i: (i, 0)))
        out_shape.append(jax.ShapeDtypeStruct((nq, n_out), jnp.float32))
    res = pl.pallas_call(
        functools.partial(_extract_cand_body, n_out, want_vals),
        grid=grid,
        in_specs=[
            pl.BlockSpec((qb, w), lambda i: (i, 0)),
            pl.BlockSpec((qb, w), lambda i: (i, 0)),
        ],
        out_specs=out_specs,
        out_shape=out_shape,
        scratch_shapes=[pltpu.VMEM((qb, w), jnp.float32)],
        compiler_params=pltpu.CompilerParams(
            dimension_semantics=("arbitrary",)),
        interpret=_INTERPRET,
    )(vals, ids)
    return res


def _gather_rows(table2d, flat_idx):
    return jnp.take(table2d, flat_idx, axis=0)


def kernel(queries, documents, Wq, bq, Wd, bd):
    n_docs = documents.shape[0]
    nq = queries.shape[0]
    ndp = ((n_docs + DB - 1) // DB) * DB
    nb = ndp // DB
    qb = 256 if nq % 256 == 0 else nq

    d_logits = documents @ Wd + bd
    q_logits = queries @ Wq + bq
    pq0 = jnp.exp(jax.nn.log_sigmoid(-q_logits))
    pq1 = jnp.exp(jax.nn.log_sigmoid(q_logits))
    pd0 = jnp.exp(jax.nn.log_sigmoid(-d_logits))
    pd1 = jnp.exp(jax.nn.log_sigmoid(d_logits))
    pad = ndp - n_docs
    pd0 = jnp.pad(pd0, ((0, pad), (0, 0)))
    pd1 = jnp.pad(pd1, ((0, pad), (0, 0)))

    j = jnp.arange(DB, dtype=jnp.int32)[:, None]
    c = jnp.arange(DB // G, dtype=jnp.int32)[None, :]
    t = jnp.arange(DB // 256, dtype=jnp.int32)[None, :]
    sel = jnp.concatenate(
        [(j == 16 * c).astype(jnp.float32), (j == 256 * t).astype(jnp.float32)],
        axis=1)

    l0, l1_3d, l2_3d = _compute_scores(pq0, pq1, pd0, pd1, sel, n_docs)
    w1 = ndp // G
    w2 = ndp // 256

    l2 = l2_3d.transpose(1, 0, 2).reshape(nq, w2)
    ids2 = _extract_l2(l2, qb)

    qrow = jnp.arange(nq, dtype=jnp.int32)[:, None]
    r1 = (ids2 >> 3) * (nq * 8) + qrow * 8 + (ids2 & 7)
    cand1 = _gather_rows(l1_3d.reshape(nb * nq * 8, G),
                         r1.reshape(-1)).reshape(nq, CAND)
    k16 = jnp.arange(G, dtype=jnp.int32)
    ids1_exp = (((ids2 >> 3) * 128 + (ids2 & 7) * 16)[:, :, None] + k16
                ).reshape(nq, CAND)
    (ids1,) = _extract_cand(cand1, ids1_exp, NSEL, False, qb)

    r0 = qrow * w1 + ids1
    cand0 = _gather_rows(l0.reshape(nq * w1, G),
                         r0.reshape(-1)).reshape(nq, CAND)
    ids0_exp = (((ids1 >> 7) * 2048 + (ids1 & 127) * 16)[:, :, None] + k16
                ).reshape(nq, CAND)
    top_ids, top_vals = _extract_cand(cand0, ids0_exp, TOP_K, True, qb)
    return top_vals, top_ids
